# baseline (speedup 1.0000x reference)
import functools

import numpy as np
import jax
import jax.numpy as jnp
from jax import lax
from jax.experimental import pallas as pl
from jax.experimental.pallas import tpu as pltpu
from jax.experimental.pallas import tpu_sc as plsc

EMBED = 128
WINDOW = 2048
BATCH = 4
TOTAL = BATCH * WINDOW
SCALE = float(np.sqrt(np.float32(EMBED)))

NC = 2
NS = 16
NW = NC * NS
BPW = TOTAL // NW
LANES = 16
NCHUNK = 4
CH = BPW // NCHUNK
PWORDS = EMBED


def _pos_encoding_packed() -> np.ndarray:
    half = EMBED // 2
    positions = np.arange(WINDOW, dtype=np.float32)[:, None]
    depths = np.arange(half, dtype=np.float32)[None, :] / np.float32(half)
    angle_rates = 1.0 / (10000.0 ** depths)
    angle_rads = positions * angle_rates
    pos = np.concatenate([np.sin(angle_rads), np.cos(angle_rads)], axis=-1)
    return pos.astype(np.float32).view(np.int32).reshape(-1)


_POS_PACKED = _pos_encoding_packed()

_mesh = plsc.VectorSubcoreMesh(core_axis_name="c", subcore_axis_name="s")


@functools.partial(
    pl.kernel,
    mesh=_mesh,
    compiler_params=pltpu.CompilerParams(use_tc_tiling_on_sc=True),
    out_type=jax.ShapeDtypeStruct((TOTAL, EMBED), jnp.float32),
    scratch_types=[
        pltpu.VMEM((BPW,), jnp.int32),
        pltpu.VMEM((BPW, EMBED), jnp.float32),
        pltpu.VMEM((BPW * PWORDS,), jnp.int32),
        pltpu.SemaphoreType.DMA,
    ]
    + [pltpu.SemaphoreType.DMA] * NCHUNK
    + [pltpu.SemaphoreType.DMA] * NCHUNK,
)
def _emb_kernel(x_hbm, table_hbm, pos_hbm, out_hbm, idx_v, rows_v, pos_v,
                sem_p, *sems):
    gsems = sems[:NCHUNK]
    wsems = sems[NCHUNK:]
    wid = lax.axis_index("s") * NC + lax.axis_index("c")
    base = wid * BPW
    b = base // WINDOW
    l0 = lax.rem(base, WINDOW)
    pos_cp = pltpu.async_copy(
        pos_hbm.at[pl.ds(l0 * PWORDS, BPW * PWORDS)], pos_v, sem_p)
    pltpu.sync_copy(x_hbm.at[b, pl.ds(l0, BPW)], idx_v)
    gcps = [
        pltpu.async_copy(
            table_hbm.at[idx_v.at[pl.ds(c * CH, CH)]],
            rows_v.at[pl.ds(c * CH, CH)],
            gsems[c])
        for c in range(NCHUNK)
    ]
    pos_cp.wait()

    wcps = []
    for c in range(NCHUNK):
        gcps[c].wait()

        def row_step(j, carry, _c=c):
            for u in range(2):
                r = _c * CH + j * 2 + u
                pbase = r * PWORDS
                for k in range(EMBED // LANES):
                    w = pos_v[pl.ds(pbase + k * LANES, LANES)]
                    p = lax.bitcast_convert_type(w, jnp.float32)
                    sl = pl.ds(k * LANES, LANES)
                    rows_v[r, sl] = rows_v[r, sl] * SCALE + p
            return carry

        lax.fori_loop(0, CH // 2, row_step, 0)
        wcps.append(pltpu.async_copy(
            rows_v.at[pl.ds(c * CH, CH)],
            out_hbm.at[pl.ds(base + c * CH, CH)],
            wsems[c]))
    for w in wcps:
        w.wait()


def kernel(x, table):
    pos = jnp.asarray(_POS_PACKED)
    out = _emb_kernel(x.astype(jnp.int32), table, pos)
    return out.reshape(BATCH, WINDOW, EMBED)


# speedup vs baseline: 1.0325x; 1.0325x over previous
import functools

import numpy as np
import jax
import jax.numpy as jnp
from jax import lax
from jax.experimental import pallas as pl
from jax.experimental.pallas import tpu as pltpu
from jax.experimental.pallas import tpu_sc as plsc

EMBED = 128
WINDOW = 2048
BATCH = 4
TOTAL = BATCH * WINDOW
SCALE = float(np.sqrt(np.float32(EMBED)))

NC = 2
NS = 16
NW = NC * NS
BPW = TOTAL // NW
LANES = 16
NCHUNK = 4
CH = BPW // NCHUNK
PWORDS = EMBED


def _pos_encoding_packed() -> np.ndarray:
    half = EMBED // 2
    positions = np.arange(WINDOW, dtype=np.float32)[:, None]
    depths = np.arange(half, dtype=np.float32)[None, :] / np.float32(half)
    angle_rates = 1.0 / (10000.0 ** depths)
    angle_rads = positions * angle_rates
    pos = np.concatenate([np.sin(angle_rads), np.cos(angle_rads)], axis=-1)
    pos = pos.astype(np.float32) / np.float32(SCALE)
    return pos.reshape(WINDOW, EMBED)


_POS_PACKED = _pos_encoding_packed()

_mesh = plsc.VectorSubcoreMesh(core_axis_name="c", subcore_axis_name="s")


@functools.partial(
    pl.kernel,
    mesh=_mesh,
    compiler_params=pltpu.CompilerParams(use_tc_tiling_on_sc=True),
    out_type=jax.ShapeDtypeStruct((TOTAL, EMBED), jnp.float32),
    scratch_types=[
        pltpu.VMEM((BPW,), jnp.int32),
        pltpu.VMEM((BPW, EMBED), jnp.float32),
        pltpu.SemaphoreType.DMA,
    ]
    + [pltpu.SemaphoreType.DMA] * NCHUNK
    + [pltpu.SemaphoreType.DMA] * NCHUNK,
)
def _emb_kernel(x_hbm, table_hbm, pos_hbm, out_hbm, idx_v, rows_v,
                sem_p, *sems):
    gsems = sems[:NCHUNK]
    wsems = sems[NCHUNK:]
    wid = lax.axis_index("s") * NC + lax.axis_index("c")
    base = wid * BPW
    b = base // WINDOW
    l0 = lax.rem(base, WINDOW)
    pos_cp = pltpu.async_copy(
        pos_hbm.at[pl.ds(l0, BPW)], rows_v, sem_p)
    pltpu.sync_copy(x_hbm.at[b, pl.ds(l0, BPW)], idx_v)
    pos_cp.wait()
    gcps = [
        pltpu.async_copy(
            table_hbm.at[idx_v.at[pl.ds(c * CH, CH)]],
            rows_v.at[pl.ds(c * CH, CH)],
            gsems[c],
            add=True)
        for c in range(NCHUNK)
    ]

    wcps = []
    for c in range(NCHUNK):
        gcps[c].wait()

        def row_step(j, carry, _c=c):
            r = _c * CH + j
            for k in range(EMBED // LANES):
                sl = pl.ds(k * LANES, LANES)
                rows_v[r, sl] = rows_v[r, sl] * SCALE
            return carry

        lax.fori_loop(0, CH, row_step, 0)
        wcps.append(pltpu.async_copy(
            rows_v.at[pl.ds(c * CH, CH)],
            out_hbm.at[pl.ds(base + c * CH, CH)],
            wsems[c]))
    for w in wcps:
        w.wait()


def kernel(x, table):
    pos = jnp.asarray(_POS_PACKED)
    out = _emb_kernel(x.astype(jnp.int32), table, pos)
    return out.reshape(BATCH, WINDOW, EMBED)


# speedup vs baseline: 1.0359x; 1.0033x over previous
import functools

import numpy as np
import jax
import jax.numpy as jnp
from jax import lax
from jax.experimental import pallas as pl
from jax.experimental.pallas import tpu as pltpu
from jax.experimental.pallas import tpu_sc as plsc

EMBED = 128
WINDOW = 2048
BATCH = 4
TOTAL = BATCH * WINDOW
SCALE = float(np.sqrt(np.float32(EMBED)))

NC = 2
NS = 16
NW = NC * NS
BPW = TOTAL // NW
LANES = 16
NCHUNK = 4
CH = BPW // NCHUNK
PWORDS = EMBED


def _pos_encoding_packed() -> np.ndarray:
    half = EMBED // 2
    positions = np.arange(WINDOW, dtype=np.float32)[:, None]
    depths = np.arange(half, dtype=np.float32)[None, :] / np.float32(half)
    angle_rates = 1.0 / (10000.0 ** depths)
    angle_rads = positions * angle_rates
    pos = np.concatenate([np.sin(angle_rads), np.cos(angle_rads)], axis=-1)
    pos = pos.astype(np.float32) / np.float32(SCALE)
    return pos.reshape(WINDOW, EMBED)


_POS_PACKED = _pos_encoding_packed()

_mesh = plsc.VectorSubcoreMesh(core_axis_name="c", subcore_axis_name="s")


@functools.partial(
    pl.kernel,
    mesh=_mesh,
    compiler_params=pltpu.CompilerParams(use_tc_tiling_on_sc=True),
    out_type=jax.ShapeDtypeStruct((TOTAL, EMBED), jnp.float32),
    scratch_types=[
        pltpu.VMEM((BPW,), jnp.int32),
        pltpu.VMEM((BPW, EMBED), jnp.float32),
        pltpu.SemaphoreType.DMA,
    ]
    + [pltpu.SemaphoreType.DMA] * NCHUNK
    + [pltpu.SemaphoreType.DMA] * NCHUNK
    + [pltpu.SemaphoreType.DMA] * NCHUNK,
)
def _emb_kernel(x_hbm, table_hbm, pos_hbm, out_hbm, idx_v, rows_v,
                sem_p, *sems):
    gsems = sems[:NCHUNK]
    wsems = sems[NCHUNK:2 * NCHUNK]
    psems = sems[2 * NCHUNK:]
    wid = lax.axis_index("s") * NC + lax.axis_index("c")
    base = wid * BPW
    b = base // WINDOW
    l0 = lax.rem(base, WINDOW)
    pcps = [
        pltpu.async_copy(
            pos_hbm.at[pl.ds(l0 + c * CH, CH)],
            rows_v.at[pl.ds(c * CH, CH)],
            psems[c])
        for c in range(NCHUNK)
    ]
    pltpu.sync_copy(x_hbm.at[b, pl.ds(l0, BPW)], idx_v)
    gcps = []
    for c in range(NCHUNK):
        pcps[c].wait()
        gcps.append(pltpu.async_copy(
            table_hbm.at[idx_v.at[pl.ds(c * CH, CH)]],
            rows_v.at[pl.ds(c * CH, CH)],
            gsems[c],
            add=True))

    wcps = []
    for c in range(NCHUNK):
        gcps[c].wait()

        def row_step(j, carry, _c=c):
            r = _c * CH + j
            for k in range(EMBED // LANES):
                sl = pl.ds(k * LANES, LANES)
                rows_v[r, sl] = rows_v[r, sl] * SCALE
            return carry

        lax.fori_loop(0, CH, row_step, 0)
        wcps.append(pltpu.async_copy(
            rows_v.at[pl.ds(c * CH, CH)],
            out_hbm.at[pl.ds(base + c * CH, CH)],
            wsems[c]))
    for w in wcps:
        w.wait()


def kernel(x, table):
    pos = jnp.asarray(_POS_PACKED)
    out = _emb_kernel(x.astype(jnp.int32), table, pos)
    return out.reshape(BATCH, WINDOW, EMBED)


# speedup vs baseline: 1.0416x; 1.0056x over previous
import functools

import numpy as np
import jax
import jax.numpy as jnp
from jax import lax
from jax.experimental import pallas as pl
from jax.experimental.pallas import tpu as pltpu
from jax.experimental.pallas import tpu_sc as plsc

EMBED = 128
WINDOW = 2048
BATCH = 4
TOTAL = BATCH * WINDOW
SCALE = float(np.sqrt(np.float32(EMBED)))

NC = 2
NS = 16
NW = NC * NS
BPW = TOTAL // NW
LANES = 16
NCHUNK = 8
CH = BPW // NCHUNK
PWORDS = EMBED


def _pos_encoding_packed() -> np.ndarray:
    half = EMBED // 2
    positions = np.arange(WINDOW, dtype=np.float32)[:, None]
    depths = np.arange(half, dtype=np.float32)[None, :] / np.float32(half)
    angle_rates = 1.0 / (10000.0 ** depths)
    angle_rads = positions * angle_rates
    pos = np.concatenate([np.sin(angle_rads), np.cos(angle_rads)], axis=-1)
    pos = pos.astype(np.float32) / np.float32(SCALE)
    return pos.reshape(WINDOW, EMBED)


_POS_PACKED = _pos_encoding_packed()

_mesh = plsc.VectorSubcoreMesh(core_axis_name="c", subcore_axis_name="s")


@functools.partial(
    pl.kernel,
    mesh=_mesh,
    compiler_params=pltpu.CompilerParams(use_tc_tiling_on_sc=True),
    out_type=jax.ShapeDtypeStruct((TOTAL, EMBED), jnp.float32),
    scratch_types=[
        pltpu.VMEM((BPW,), jnp.int32),
        pltpu.VMEM((BPW, EMBED), jnp.float32),
        pltpu.SemaphoreType.DMA,
    ]
    + [pltpu.SemaphoreType.DMA] * NCHUNK
    + [pltpu.SemaphoreType.DMA] * NCHUNK
    + [pltpu.SemaphoreType.DMA] * NCHUNK,
)
def _emb_kernel(x_hbm, table_hbm, pos_hbm, out_hbm, idx_v, rows_v,
                sem_p, *sems):
    gsems = sems[:NCHUNK]
    wsems = sems[NCHUNK:2 * NCHUNK]
    psems = sems[2 * NCHUNK:]
    wid = lax.axis_index("s") * NC + lax.axis_index("c")
    base = wid * BPW
    b = base // WINDOW
    l0 = lax.rem(base, WINDOW)
    pcps = [
        pltpu.async_copy(
            pos_hbm.at[pl.ds(l0 + c * CH, CH)],
            rows_v.at[pl.ds(c * CH, CH)],
            psems[c])
        for c in range(NCHUNK)
    ]
    pltpu.sync_copy(x_hbm.at[b, pl.ds(l0, BPW)], idx_v)
    gcps = []
    for c in range(NCHUNK):
        pcps[c].wait()
        gcps.append(pltpu.async_copy(
            table_hbm.at[idx_v.at[pl.ds(c * CH, CH)]],
            rows_v.at[pl.ds(c * CH, CH)],
            gsems[c],
            add=True))

    wcps = []
    for c in range(NCHUNK):
        gcps[c].wait()

        def row_step(j, carry, _c=c):
            r = _c * CH + j
            for k in range(EMBED // LANES):
                sl = pl.ds(k * LANES, LANES)
                rows_v[r, sl] = rows_v[r, sl] * SCALE
            return carry

        lax.fori_loop(0, CH, row_step, 0)
        wcps.append(pltpu.async_copy(
            rows_v.at[pl.ds(c * CH, CH)],
            out_hbm.at[pl.ds(base + c * CH, CH)],
            wsems[c]))
    for w in wcps:
        w.wait()


def kernel(x, table):
    pos = jnp.asarray(_POS_PACKED)
    out = _emb_kernel(x.astype(jnp.int32), table, pos)
    return out.reshape(BATCH, WINDOW, EMBED)
